# Initial kernel scaffold; baseline (speedup 1.0000x reference)
import jax
import jax.numpy as jnp
from jax.experimental import pallas as pl


def kernel(x, edge_index, W1, b1, W2, b2):
    raise NotImplementedError("write your pallas kernel here")


# speedup vs baseline: 11.8882x; 11.8882x over previous
import jax
import jax.numpy as jnp
from jax import lax
from jax.experimental import pallas as pl
from jax.experimental.pallas import tpu as pltpu
from jax.experimental.pallas import tpu_sc as plsc

NC = 2
NS = 16
NW = NC * NS
D = 128
BN = 1000


def _edge_chunk(per_w):
    for k in range(128, 0, -8):
        if per_w % k == 0:
            return k
    raise ValueError(per_w)


def _round_up(v, m):
    return (v + m - 1) // m * m


def _sc_degree(dst, n):
    e = dst.shape[0]
    per_w = e // NW
    ck = _edge_chunk(per_w)
    iters = per_w // ck
    rpt = _round_up(-(-n // NS), 8)
    np_ = rpt * NS
    mesh = plsc.VectorSubcoreMesh(core_axis_name="c", subcore_axis_name="s")

    def body(dst_h, zeros_h, ones_h, out_h, idx_d, ones_v, acc_sh, sem):
        c = lax.axis_index("c")
        s = lax.axis_index("s")
        wid = s * NC + c

        base_r = s * rpt
        pltpu.sync_copy(zeros_h, acc_sh.at[pl.ds(base_r, rpt)])
        pltpu.sync_copy(ones_h, ones_v)
        plsc.subcore_barrier()

        ebase = wid * per_w

        def step(i, _):
            pltpu.sync_copy(dst_h.at[pl.ds(ebase + i * ck, ck)], idx_d)
            pltpu.sync_copy(ones_v, acc_sh.at[idx_d], add=True)
            return 0
        lax.fori_loop(0, iters, step, 0)
        plsc.subcore_barrier()
        pltpu.sync_copy(acc_sh.at[pl.ds(base_r, rpt)],
                        out_h.at[pl.ds(c * np_ + base_r, rpt)])

    return pl.kernel(
        body,
        out_type=jax.ShapeDtypeStruct((NC * np_, D), jnp.float32),
        mesh=mesh,
        scratch_types=[
            pltpu.VMEM((ck,), jnp.int32),
            pltpu.VMEM((ck, D), jnp.float32),
            pltpu.VMEM_SHARED((np_, D), jnp.float32),
            pltpu.SemaphoreType.DMA,
        ],
    )(dst, jnp.zeros((rpt, D), jnp.float32),
      jnp.ones((ck, D), jnp.float32)), np_


def _sc_scatter(table, src, dst):
    n = table.shape[0]
    e = src.shape[0]
    per_w = e // NW
    ck = _edge_chunk(per_w)
    iters = per_w // ck
    rpt = _round_up(-(-n // NS), 8)
    np_ = rpt * NS
    mesh = plsc.VectorSubcoreMesh(core_axis_name="c", subcore_axis_name="s")

    def body(table_h, src_h, dst_h, zeros_h, out_h, idx_s, idx_d, rows_v,
             acc_sh, sem):
        c = lax.axis_index("c")
        s = lax.axis_index("s")
        wid = s * NC + c

        base_r = s * rpt
        pltpu.sync_copy(zeros_h, acc_sh.at[pl.ds(base_r, rpt)])
        plsc.subcore_barrier()

        ebase = wid * per_w

        def step(i, _):
            off = ebase + i * ck
            pltpu.sync_copy(src_h.at[pl.ds(off, ck)], idx_s)
            pltpu.sync_copy(dst_h.at[pl.ds(off, ck)], idx_d)
            pltpu.async_copy(table_h.at[idx_s], rows_v, sem).wait()
            pltpu.sync_copy(rows_v, acc_sh.at[idx_d], add=True)
            return 0
        lax.fori_loop(0, iters, step, 0)
        plsc.subcore_barrier()
        pltpu.sync_copy(acc_sh.at[pl.ds(base_r, rpt)],
                        out_h.at[pl.ds(c * np_ + base_r, rpt)])

    return pl.kernel(
        body,
        out_type=jax.ShapeDtypeStruct((NC * np_, D), jnp.float32),
        mesh=mesh,
        scratch_types=[
            pltpu.VMEM((ck,), jnp.int32),
            pltpu.VMEM((ck,), jnp.int32),
            pltpu.VMEM((ck, D), jnp.float32),
            pltpu.VMEM_SHARED((np_, D), jnp.float32),
            pltpu.SemaphoreType.DMA,
        ],
    )(table, src, dst, jnp.zeros((rpt, D), jnp.float32)), np_


def _tc_stage1(x, w1, d0, d1):
    n = x.shape[0]

    def body(x_r, w_r, d0_r, d1_r, h_r, hh_r, dinv_r):
        deg = d0_r[...] + d1_r[...] + 1.0
        dinv = lax.rsqrt(deg)
        h = jnp.dot(x_r[...], w_r[...], preferred_element_type=jnp.float32)
        h_r[...] = h
        hh_r[...] = h * dinv
        dinv_r[...] = dinv

    return pl.pallas_call(
        body,
        grid=(n // BN,),
        in_specs=[
            pl.BlockSpec((BN, D), lambda i: (i, 0)),
            pl.BlockSpec((D, D), lambda i: (0, 0)),
            pl.BlockSpec((BN, 1), lambda i: (i, 0)),
            pl.BlockSpec((BN, 1), lambda i: (i, 0)),
        ],
        out_specs=[
            pl.BlockSpec((BN, D), lambda i: (i, 0)),
            pl.BlockSpec((BN, D), lambda i: (i, 0)),
            pl.BlockSpec((BN, 1), lambda i: (i, 0)),
        ],
        out_shape=[
            jax.ShapeDtypeStruct((n, D), jnp.float32),
            jax.ShapeDtypeStruct((n, D), jnp.float32),
            jax.ShapeDtypeStruct((n, 1), jnp.float32),
        ],
    )(x, w1, d0, d1)


def _tc_stage2(a0, a1, h1, dinv, w2, b1):
    n = h1.shape[0]

    def body(a0_r, a1_r, h1_r, dinv_r, w_r, b_r, h2_r, hh2_r):
        dinv = dinv_r[...]
        t = (a0_r[...] + a1_r[...]) * dinv + h1_r[...] * (dinv * dinv) + b_r[...]
        t = jnp.maximum(t, 0.0)
        h2 = jnp.dot(t, w_r[...], preferred_element_type=jnp.float32)
        h2_r[...] = h2
        hh2_r[...] = h2 * dinv

    return pl.pallas_call(
        body,
        grid=(n // BN,),
        in_specs=[
            pl.BlockSpec((BN, D), lambda i: (i, 0)),
            pl.BlockSpec((BN, D), lambda i: (i, 0)),
            pl.BlockSpec((BN, D), lambda i: (i, 0)),
            pl.BlockSpec((BN, 1), lambda i: (i, 0)),
            pl.BlockSpec((D, D), lambda i: (0, 0)),
            pl.BlockSpec((1, D), lambda i: (0, 0)),
        ],
        out_specs=[
            pl.BlockSpec((BN, D), lambda i: (i, 0)),
            pl.BlockSpec((BN, D), lambda i: (i, 0)),
        ],
        out_shape=[
            jax.ShapeDtypeStruct((n, D), jnp.float32),
            jax.ShapeDtypeStruct((n, D), jnp.float32),
        ],
    )(a0, a1, h1, dinv, w2, b1.reshape(1, D))


def _tc_stage3(a0, a1, h2, dinv, b2):
    n = h2.shape[0]

    def body(a0_r, a1_r, h2_r, dinv_r, b_r, o_r):
        dinv = dinv_r[...]
        o_r[...] = ((a0_r[...] + a1_r[...]) * dinv
                    + h2_r[...] * (dinv * dinv) + b_r[...])

    return pl.pallas_call(
        body,
        grid=(n // BN,),
        in_specs=[
            pl.BlockSpec((BN, D), lambda i: (i, 0)),
            pl.BlockSpec((BN, D), lambda i: (i, 0)),
            pl.BlockSpec((BN, D), lambda i: (i, 0)),
            pl.BlockSpec((BN, 1), lambda i: (i, 0)),
            pl.BlockSpec((1, D), lambda i: (0, 0)),
        ],
        out_specs=pl.BlockSpec((BN, D), lambda i: (i, 0)),
        out_shape=jax.ShapeDtypeStruct((n, D), jnp.float32),
    )(a0, a1, h2, dinv, b2.reshape(1, D))


def kernel(x, edge_index, W1, b1, W2, b2):
    n = x.shape[0]
    src = edge_index[0]
    dst = edge_index[1]

    degp, np_ = _sc_degree(dst, n)
    d0 = degp[:n, 0:1]
    d1 = degp[np_:np_ + n, 0:1]
    h1, hh1, dinv = _tc_stage1(x, W1, d0, d1)
    agg1, _ = _sc_scatter(hh1, src, dst)
    h2, hh2 = _tc_stage2(agg1[:n], agg1[np_:np_ + n], h1, dinv, W2, b1)
    agg2, _ = _sc_scatter(hh2, src, dst)
    return _tc_stage3(agg2[:n], agg2[np_:np_ + n], h2, dinv, b2)


# speedup vs baseline: 14.4472x; 1.2153x over previous
import jax
import jax.numpy as jnp
from jax import lax
from jax.experimental import pallas as pl
from jax.experimental.pallas import tpu as pltpu
from jax.experimental.pallas import tpu_sc as plsc

NC = 2
NS = 16
NW = NC * NS
D = 128
BN = 1000


def _edge_chunk(per_w, maxk=128):
    for k in range(maxk, 0, -8):
        if per_w % k == 0:
            return k
    raise ValueError(per_w)


def _round_up(v, m):
    return (v + m - 1) // m * m


def _sc_degree(dst, n):
    e = dst.shape[0]
    per_w = e // NW
    ck = _edge_chunk(per_w)
    iters = per_w // ck
    rpt = _round_up(-(-n // NS), 8)
    np_ = rpt * NS
    mesh = plsc.VectorSubcoreMesh(core_axis_name="c", subcore_axis_name="s")

    def body(dst_h, zeros_h, ones_h, out_h, idx_v, ones_v, acc_sh, sem):
        c = lax.axis_index("c")
        s = lax.axis_index("s")
        wid = s * NC + c

        base_r = s * rpt
        pltpu.sync_copy(zeros_h, acc_sh.at[pl.ds(base_r, rpt)])
        pltpu.sync_copy(ones_h, ones_v)
        pltpu.sync_copy(dst_h.at[wid], idx_v)
        plsc.subcore_barrier()

        def step(i, _):
            pltpu.sync_copy(ones_v, acc_sh.at[idx_v.at[i]], add=True)
            return 0
        lax.fori_loop(0, iters, step, 0)
        plsc.subcore_barrier()
        pltpu.sync_copy(acc_sh.at[pl.ds(base_r, rpt)],
                        out_h.at[pl.ds(c * np_ + base_r, rpt)])

    return pl.kernel(
        body,
        out_type=jax.ShapeDtypeStruct((NC * np_, D), jnp.float32),
        mesh=mesh,
        scratch_types=[
            pltpu.VMEM((iters, ck), jnp.int32),
            pltpu.VMEM((ck, D), jnp.float32),
            pltpu.VMEM_SHARED((np_, D), jnp.float32),
            pltpu.SemaphoreType.DMA,
        ],
    )(dst.reshape(NW, iters, ck), jnp.zeros((rpt, D), jnp.float32),
      jnp.ones((ck, D), jnp.float32)), np_


def _sc_scatter(table, src, dst):
    n = table.shape[0]
    e = src.shape[0]
    per_w = e // NW
    ck = 128
    nck = -(-per_w // ck)
    pe = nck * ck
    rpt = _round_up(-(-n // NS), 8)
    np_ = rpt * NS
    assert per_w == pe or n < np_
    PH = 2
    L = -(-nck // PH)
    mesh = plsc.VectorSubcoreMesh(core_axis_name="c", subcore_axis_name="s")

    def body(table_h, src_h, dst_h, zeros_h, out_h, idx_s, idx_d,
             rows_a, rows_b, acc_sh, sem_a, sem_b):
        c = lax.axis_index("c")
        s = lax.axis_index("s")
        wid = s * NC + c

        base_r = s * rpt
        pltpu.sync_copy(zeros_h, acc_sh.at[pl.ds(base_r, rpt)])
        plsc.subcore_barrier()

        def gather(i, buf, sem):
            pltpu.async_copy(table_h.at[idx_s.at[i]], buf, sem)

        def gwait(buf, sem):
            pltpu.make_async_copy(table_h.at[pl.ds(0, ck)], buf, sem).wait()

        for p in range(PH):
            cbase = p * L
            lp = min(L, nck - cbase)
            if lp <= 0:
                continue
            pltpu.sync_copy(src_h.at[wid, pl.ds(cbase, lp)],
                            idx_s.at[pl.ds(0, lp)])
            pltpu.sync_copy(dst_h.at[wid, pl.ds(cbase, lp)],
                            idx_d.at[pl.ds(0, lp)])

            gather(0, rows_a, sem_a)

            def pair(k, _):
                i = 2 * k
                gwait(rows_a, sem_a)
                gather(i + 1, rows_b, sem_b)
                pltpu.sync_copy(rows_a, acc_sh.at[idx_d.at[i]], add=True)
                gwait(rows_b, sem_b)

                @pl.when(i + 2 < lp)
                def _():
                    gather(i + 2, rows_a, sem_a)
                pltpu.sync_copy(rows_b, acc_sh.at[idx_d.at[i + 1]], add=True)
                return 0
            lax.fori_loop(0, lp // 2, pair, 0)
            if lp % 2:
                gwait(rows_a, sem_a)
                pltpu.sync_copy(rows_a, acc_sh.at[idx_d.at[lp - 1]], add=True)

        plsc.subcore_barrier()
        pltpu.sync_copy(acc_sh.at[pl.ds(base_r, rpt)],
                        out_h.at[pl.ds(c * np_ + base_r, rpt)])

    pad = pe - per_w
    srcp = jnp.pad(src.reshape(NW, per_w), ((0, 0), (0, pad)))
    dstp = jnp.pad(dst.reshape(NW, per_w), ((0, 0), (0, pad)),
                   constant_values=n)
    return pl.kernel(
        body,
        out_type=jax.ShapeDtypeStruct((NC * np_, D), jnp.float32),
        mesh=mesh,
        scratch_types=[
            pltpu.VMEM((L, ck), jnp.int32),
            pltpu.VMEM((L, ck), jnp.int32),
            pltpu.VMEM((ck, D), jnp.float32),
            pltpu.VMEM((ck, D), jnp.float32),
            pltpu.VMEM_SHARED((np_, D), jnp.float32),
            pltpu.SemaphoreType.DMA,
            pltpu.SemaphoreType.DMA,
        ],
    )(table, srcp.reshape(NW, nck, ck), dstp.reshape(NW, nck, ck),
      jnp.zeros((rpt, D), jnp.float32)), np_


def _tc_stage1(x, w1, d0, d1):
    n = x.shape[0]

    def body(x_r, w_r, d0_r, d1_r, h_r, hh_r, dinv_r):
        deg = d0_r[...] + d1_r[...] + 1.0
        dinv = lax.rsqrt(deg)
        h = jnp.dot(x_r[...], w_r[...], preferred_element_type=jnp.float32)
        h_r[...] = h
        hh_r[...] = h * dinv
        dinv_r[...] = dinv

    return pl.pallas_call(
        body,
        grid=(n // BN,),
        in_specs=[
            pl.BlockSpec((BN, D), lambda i: (i, 0)),
            pl.BlockSpec((D, D), lambda i: (0, 0)),
            pl.BlockSpec((BN, 1), lambda i: (i, 0)),
            pl.BlockSpec((BN, 1), lambda i: (i, 0)),
        ],
        out_specs=[
            pl.BlockSpec((BN, D), lambda i: (i, 0)),
            pl.BlockSpec((BN, D), lambda i: (i, 0)),
            pl.BlockSpec((BN, 1), lambda i: (i, 0)),
        ],
        out_shape=[
            jax.ShapeDtypeStruct((n, D), jnp.float32),
            jax.ShapeDtypeStruct((n, D), jnp.float32),
            jax.ShapeDtypeStruct((n, 1), jnp.float32),
        ],
    )(x, w1, d0, d1)


def _tc_stage2(a0, a1, h1, dinv, w2, b1):
    n = h1.shape[0]

    def body(a0_r, a1_r, h1_r, dinv_r, w_r, b_r, h2_r, hh2_r):
        dinv = dinv_r[...]
        t = (a0_r[...] + a1_r[...]) * dinv + h1_r[...] * (dinv * dinv) + b_r[...]
        t = jnp.maximum(t, 0.0)
        h2 = jnp.dot(t, w_r[...], preferred_element_type=jnp.float32)
        h2_r[...] = h2
        hh2_r[...] = h2 * dinv

    return pl.pallas_call(
        body,
        grid=(n // BN,),
        in_specs=[
            pl.BlockSpec((BN, D), lambda i: (i, 0)),
            pl.BlockSpec((BN, D), lambda i: (i, 0)),
            pl.BlockSpec((BN, D), lambda i: (i, 0)),
            pl.BlockSpec((BN, 1), lambda i: (i, 0)),
            pl.BlockSpec((D, D), lambda i: (0, 0)),
            pl.BlockSpec((1, D), lambda i: (0, 0)),
        ],
        out_specs=[
            pl.BlockSpec((BN, D), lambda i: (i, 0)),
            pl.BlockSpec((BN, D), lambda i: (i, 0)),
        ],
        out_shape=[
            jax.ShapeDtypeStruct((n, D), jnp.float32),
            jax.ShapeDtypeStruct((n, D), jnp.float32),
        ],
    )(a0, a1, h1, dinv, w2, b1.reshape(1, D))


def _tc_stage3(a0, a1, h2, dinv, b2):
    n = h2.shape[0]

    def body(a0_r, a1_r, h2_r, dinv_r, b_r, o_r):
        dinv = dinv_r[...]
        o_r[...] = ((a0_r[...] + a1_r[...]) * dinv
                    + h2_r[...] * (dinv * dinv) + b_r[...])

    return pl.pallas_call(
        body,
        grid=(n // BN,),
        in_specs=[
            pl.BlockSpec((BN, D), lambda i: (i, 0)),
            pl.BlockSpec((BN, D), lambda i: (i, 0)),
            pl.BlockSpec((BN, D), lambda i: (i, 0)),
            pl.BlockSpec((BN, 1), lambda i: (i, 0)),
            pl.BlockSpec((1, D), lambda i: (0, 0)),
        ],
        out_specs=pl.BlockSpec((BN, D), lambda i: (i, 0)),
        out_shape=jax.ShapeDtypeStruct((n, D), jnp.float32),
    )(a0, a1, h2, dinv, b2.reshape(1, D))


def kernel(x, edge_index, W1, b1, W2, b2):
    n = x.shape[0]
    src = edge_index[0]
    dst = edge_index[1]

    degp, np_ = _sc_degree(dst, n)
    d0 = degp[:n, 0:1]
    d1 = degp[np_:np_ + n, 0:1]
    h1, hh1, dinv = _tc_stage1(x, W1, d0, d1)
    agg1, _ = _sc_scatter(hh1, src, dst)
    h2, hh2 = _tc_stage2(agg1[:n], agg1[np_:np_ + n], h1, dinv, W2, b1)
    agg2, _ = _sc_scatter(hh2, src, dst)
    return _tc_stage3(agg2[:n], agg2[np_:np_ + n], h2, dinv, b2)
